# Initial kernel scaffold; baseline (speedup 1.0000x reference)
import jax
import jax.numpy as jnp
from jax.experimental import pallas as pl


def kernel(x, embeddings, Wr, Wi, Wj, Wk, br, bi, bj, bk, W1, b1, W2, b2):
    raise NotImplementedError("write your pallas kernel here")


# speedup vs baseline: 322.4752x; 322.4752x over previous
import functools
import numpy as np
import jax
import jax.numpy as jnp
from jax import lax
from jax.experimental import pallas as pl
from jax.experimental.pallas import tpu as pltpu
from jax.experimental.pallas import tpu_sc as plsc

B = 16; SEQ = 1536; D = 32; EMB = 32; HID = 128; PRED = 96; TOPM = 32; NW = 2
WIN = 1024; HOP = 512; NF = 513; N = 1024; FPAD = 640
NBD = B * D

_t = np.arange(N)[:, None]
_f = np.arange(FPAD)[None, :]
_ang = 2.0 * np.pi * (_t * _f) / N
COS = np.where(_f < NF, np.cos(_ang), 0.0).astype(np.float32)
SIN = np.where(_f < NF, np.sin(_ang), 0.0).astype(np.float32)
_alpha = np.where((np.arange(FPAD) % 512 == 0) & (np.arange(FPAD) < NF), 1.0, 2.0)
ALPHA = (np.where(np.arange(FPAD) < NF, _alpha, 0.0) / N).astype(np.float32)
CSA = np.stack([
    (COS * ALPHA[None, :]).reshape(2, 512, FPAD),
    (SIN * ALPHA[None, :]).reshape(2, 512, FPAD),
])
QMAP = np.array([[0, 1, 2, 3, 4, 1, 0, 3, 2, 5],
                 [2, 3, 0, 1, 6, 3, 2, 1, 0, 7]], dtype=np.int32)
CSIGN = np.array([[1., -1., -1., -1., 1.], [1., -1., 1., 1., 1.]], np.float32)
SSIGN = np.array([[-1., -1., -1., 1., -1.], [-1., -1., 1., -1., -1.]], np.float32)

_F32 = jnp.float32


def _k1_body(xw_ref, cos_ref, sin_ref, fre_ref, fim_ref, amp_ref):
    xb = xw_ref[0]
    dn = (((0,), (0,)), ((), ()))
    fre = lax.dot_general(xb, cos_ref[...], dn, preferred_element_type=_F32, precision=lax.Precision.HIGHEST)
    fim = -lax.dot_general(xb, sin_ref[...], dn, preferred_element_type=_F32, precision=lax.Precision.HIGHEST)
    fre_ref[0] = fre
    fim_ref[0] = fim
    col = lax.broadcasted_iota(jnp.int32, (D, FPAD), 1)
    amp_ref[0] = jnp.where(col < NF, fre * fre + fim * fim, -1.0)


def _k1(xw, cos, sin):
    g = B * NW
    return pl.pallas_call(
        _k1_body,
        grid=(g,),
        in_specs=[
            pl.BlockSpec((1, N, D), lambda i: (i, 0, 0)),
            pl.BlockSpec((N, FPAD), lambda i: (0, 0)),
            pl.BlockSpec((N, FPAD), lambda i: (0, 0)),
        ],
        out_specs=[
            pl.BlockSpec((1, D, FPAD), lambda i: (i, 0, 0)),
            pl.BlockSpec((1, D, FPAD), lambda i: (i, 0, 0)),
            pl.BlockSpec((1, D, FPAD), lambda i: (i, 0, 0)),
        ],
        out_shape=[jax.ShapeDtypeStruct((g, D, FPAD), _F32)] * 3,
    )(xw, cos, sin)


def _k2a_body(amp_ref, fre_ref, fim_ref, idx_ref, val_ref, asc_ref):
    asc_ref[0] = amp_ref[0, 0]
    asc_ref[1] = amp_ref[0, 1]
    f0r = fre_ref[0, 0]
    f0i = fim_ref[0, 0]
    f1r = fre_ref[0, 1]
    f1i = fim_ref[0, 1]
    iota = lax.broadcasted_iota(jnp.int32, (D, FPAD), 1)
    lane64 = lax.broadcasted_iota(jnp.int32, (D, 2 * TOPM), 1)
    lane128 = lax.broadcasted_iota(jnp.int32, (D, 4 * TOPM), 1)

    def body(m, carry):
        idxs, vals = carry
        a0 = asc_ref[0]
        a1 = asc_ref[1]
        mx0 = jnp.max(a0, axis=1, keepdims=True)
        mx1 = jnp.max(a1, axis=1, keepdims=True)
        i0 = jnp.min(jnp.where(a0 == mx0, iota, FPAD), axis=1, keepdims=True)
        i1 = jnp.min(jnp.where(a1 == mx1, iota, FPAD), axis=1, keepdims=True)
        oh0 = (iota == i0)
        oh1 = (iota == i1)
        pv = jnp.sum(jnp.where(oh0, f0r, 0.0), axis=1, keepdims=True)
        qv = jnp.sum(jnp.where(oh0, f0i, 0.0), axis=1, keepdims=True)
        rv = jnp.sum(jnp.where(oh1, f1r, 0.0), axis=1, keepdims=True)
        sv = jnp.sum(jnp.where(oh1, f1i, 0.0), axis=1, keepdims=True)
        idxs = jnp.where(lane64 == m, i0, idxs)
        idxs = jnp.where(lane64 == TOPM + m, i1, idxs)
        vals = jnp.where(lane128 == m, pv, vals)
        vals = jnp.where(lane128 == TOPM + m, qv, vals)
        vals = jnp.where(lane128 == 2 * TOPM + m, rv, vals)
        vals = jnp.where(lane128 == 3 * TOPM + m, sv, vals)
        asc_ref[0] = jnp.where(oh0, -1.0, a0)
        asc_ref[1] = jnp.where(oh1, -1.0, a1)
        return idxs, vals

    idxs0 = jnp.zeros((D, 2 * TOPM), jnp.int32)
    vals0 = jnp.zeros((D, 4 * TOPM), _F32)
    idxs, vals = lax.fori_loop(0, TOPM, body, (idxs0, vals0))
    idx_ref[0] = idxs

    pv_ = vals[:, :TOPM]
    qv_ = vals[:, TOPM:2 * TOPM]
    rv_ = vals[:, 2 * TOPM:3 * TOPM]
    sv_ = vals[:, 3 * TOPM:]
    val_ref[0] = jnp.concatenate(
        [pv_, qv_, rv_, sv_, pv_, qv_, rv_, sv_], axis=1)


def _k2a(amp, fre, fim):
    return pl.pallas_call(
        _k2a_body,
        grid=(B,),
        in_specs=[pl.BlockSpec((1, NW, D, FPAD), lambda i: (i, 0, 0, 0))] * 3,
        out_specs=[pl.BlockSpec((1, D, 2 * TOPM), lambda i: (i, 0, 0)),
                   pl.BlockSpec((1, D, 8 * TOPM), lambda i: (i, 0, 0))],
        out_shape=[jax.ShapeDtypeStruct((B, D, 2 * TOPM), jnp.int32),
                   jax.ShapeDtypeStruct((B, D, 8 * TOPM), _F32)],
        scratch_shapes=[pltpu.VMEM((2, D, FPAD), _F32)],
    )(amp, fre, fim)


_SC_ROWS = NBD // 32


def _k2b_body(idx_hbm, val_hbm, out_hbm, vbuf, idxv, valv, zero16):
    nc = 2
    wid = lax.axis_index("s") * nc + lax.axis_index("c")
    zero16[...] = jnp.zeros((16,), _F32)
    for c in range(10 * FPAD // 16):
        vbuf[pl.ds(c * 16, 16)] = zero16[...]

    def row_body(r, _):
        row = wid * _SC_ROWS + r
        pltpu.sync_copy(idx_hbm.at[row], idxv)
        pltpu.sync_copy(val_hbm.at[row], valv)
        ones = jnp.full((16,), 1.0, _F32)
        for half in range(2):
            i0 = idxv[pl.ds(half * 16, 16)]
            i1 = idxv[pl.ds(TOPM + half * 16, 16)]
            pvA = valv[pl.ds(half * 16, 16)]
            qvA = valv[pl.ds(TOPM + half * 16, 16)]
            rvA = valv[pl.ds(2 * TOPM + half * 16, 16)]
            svA = valv[pl.ds(3 * TOPM + half * 16, 16)]
            pvB = valv[pl.ds(4 * TOPM + half * 16, 16)]
            qvB = valv[pl.ds(5 * TOPM + half * 16, 16)]
            rvB = valv[pl.ds(6 * TOPM + half * 16, 16)]
            svB = valv[pl.ds(7 * TOPM + half * 16, 16)]
            for k, v in ((0, pvA), (1, qvA), (2, rvA), (3, svA), (4, ones)):
                plsc.store_scatter(vbuf, [i0 + k * FPAD], v)
            for k, v in ((5, pvB), (6, qvB), (7, rvB), (8, svB), (9, ones)):
                plsc.store_scatter(vbuf, [i1 + k * FPAD], v)
        pltpu.sync_copy(vbuf, out_hbm.at[row])
        z = jnp.zeros((16,), _F32)
        for half in range(2):
            i0 = idxv[pl.ds(half * 16, 16)]
            i1 = idxv[pl.ds(TOPM + half * 16, 16)]
            for k in range(5):
                plsc.store_scatter(vbuf, [i0 + k * FPAD], z)
            for k in range(5, 10):
                plsc.store_scatter(vbuf, [i1 + k * FPAD], z)
        return 0

    lax.fori_loop(0, _SC_ROWS, row_body, 0)


def _k2b(idx2, val2):
    mesh = plsc.VectorSubcoreMesh(core_axis_name="c", subcore_axis_name="s")
    f = pl.kernel(
        _k2b_body,
        mesh=mesh,
        out_type=jax.ShapeDtypeStruct((NBD, 10 * FPAD), _F32),
        scratch_types=[pltpu.VMEM((10 * FPAD,), _F32),
                       pltpu.VMEM((2 * TOPM,), jnp.int32),
                       pltpu.VMEM((8 * TOPM,), _F32),
                       pltpu.VMEM((16,), _F32)],
        compiler_params=pltpu.CompilerParams(needs_layout_passes=False),
    )
    return f(idx2, val2)


def _k3a_body(emb_ref, wr_ref, wi_ref, wj_ref, wk_ref,
              br_ref, bi_ref, bj_ref, bk_ref, w1t_ref, w1u_ref):
    e1 = emb_ref[...]
    dn = (((1,), (0,)), ((), ()))
    rows = [lax.dot_general(e1, wr_ref[...], dn, preferred_element_type=_F32, precision=lax.Precision.HIGHEST),
            lax.dot_general(e1, wi_ref[...], dn, preferred_element_type=_F32, precision=lax.Precision.HIGHEST),
            lax.dot_general(e1, wj_ref[...], dn, preferred_element_type=_F32, precision=lax.Precision.HIGHEST),
            lax.dot_general(e1, wk_ref[...], dn, preferred_element_type=_F32, precision=lax.Precision.HIGHEST),
            br_ref[...], bi_ref[...], bj_ref[...], bk_ref[...]]
    u = jnp.concatenate(rows, axis=0)
    w1u_ref[...] = lax.dot_general(u, w1t_ref[...], dn,
                                   preferred_element_type=_F32, precision=lax.Precision.HIGHEST)


def _k3a(emb1, Wr, Wi, Wj, Wk, br1, bi1, bj1, bk1, w1t):
    nb = 12
    blk = (SEQ * HID) // nb
    return pl.pallas_call(
        _k3a_body,
        grid=(nb,),
        in_specs=[pl.BlockSpec((1, EMB), lambda i: (0, 0))] +
                 [pl.BlockSpec((EMB, EMB), lambda i: (0, 0))] * 4 +
                 [pl.BlockSpec((1, EMB), lambda i: (0, 0))] * 4 +
                 [pl.BlockSpec((EMB, blk), lambda i: (0, i))],
        out_specs=pl.BlockSpec((8, blk), lambda i: (0, i)),
        out_shape=jax.ShapeDtypeStruct((8, SEQ * HID), _F32),
    )(emb1, Wr, Wi, Wj, Wk, br1, bi1, bj1, bk1, w1t)


def _k3b_body2(csa_ref, w1u_ref, out_ref):
    w = pl.program_id(0)
    half = pl.program_id(2)
    scale = jnp.where((w + half) % 2 == 0, 1.0, 0.5)
    a = w1u_ref[0, 0] * scale
    dn = (((0,), (0,)), ((), ()))
    part = lax.dot_general(csa_ref[0, 0], a, dn, preferred_element_type=_F32, precision=lax.Precision.HIGHEST)

    @pl.when(half == 0)
    def _():
        out_ref[0, 0] = part

    @pl.when(half != 0)
    def _():
        out_ref[0, 0] += part


def _qsel(w, jj):
    trig = jj // 5
    qpos = jj % 5
    c = 2 * w + trig
    return jnp.where(qpos < 4, jnp.bitwise_xor(qpos, c), 4 + c)


def _k3b2(csa, w1u4):
    return pl.pallas_call(
        _k3b_body2,
        grid=(2, 10, 2),
        in_specs=[
            pl.BlockSpec((1, 1, 512, FPAD),
                         lambda w, jj, h: (jj // 5, h, 0, 0)),
            pl.BlockSpec((1, 1, 512, HID),
                         lambda w, jj, h: (_qsel(w, jj), w + h, 0, 0)),
        ],
        out_specs=pl.BlockSpec((1, 1, FPAD, HID),
                               lambda w, jj, h: (w, jj, 0, 0)),
        out_shape=jax.ShapeDtypeStruct((2, 10, FPAD, HID), _F32),
    )(csa, w1u4)


def _k4_body(v_ref, t_ref, b1_ref, w2_ref, b2_ref, out_ref):
    dn = (((1,), (0,)), ((), ()))
    hpre = lax.dot_general(v_ref[...], t_ref[...], dn,
                           preferred_element_type=_F32, precision=lax.Precision.HIGHEST) + b1_ref[...]
    h = jnp.where(hpre >= 0.0, hpre, 0.01 * hpre)
    out_ref[...] = lax.dot_general(h, w2_ref[...], dn,
                                   preferred_element_type=_F32, precision=lax.Precision.HIGHEST) + b2_ref[...]


def _k4(vflat, tflat, b1, W2, b2):
    return pl.pallas_call(
        _k4_body,
        grid=(1,),
        in_specs=[
            pl.BlockSpec((NBD, 10 * FPAD), lambda i: (0, 0)),
            pl.BlockSpec((10 * FPAD, HID), lambda i: (0, 0)),
            pl.BlockSpec((1, HID), lambda i: (0, 0)),
            pl.BlockSpec((HID, PRED), lambda i: (0, 0)),
            pl.BlockSpec((1, PRED), lambda i: (0, 0)),
        ],
        out_specs=pl.BlockSpec((NBD, PRED), lambda i: (0, 0)),
        out_shape=jax.ShapeDtypeStruct((NBD, PRED), _F32),
    )(vflat, tflat, b1, W2, b2)


def kernel(x, embeddings, Wr, Wi, Wj, Wk, br, bi, bj, bk, W1, b1, W2, b2):
    cos, sin, csa = jnp.asarray(COS), jnp.asarray(SIN), jnp.asarray(CSA)
    xw = jnp.stack([x[:, :N, :], x[:, HOP:HOP + N, :]], axis=1)
    xw = xw.reshape(B * NW, N, D)
    fre, fim, amp = _k1(xw, cos, sin)
    fre = fre.reshape(B, NW, D, FPAD)
    fim = fim.reshape(B, NW, D, FPAD)
    amp = amp.reshape(B, NW, D, FPAD)
    idx2, val2 = _k2a(amp, fre, fim)
    vflat = _k2b(idx2.reshape(NBD, 2 * TOPM), val2.reshape(NBD, 8 * TOPM))
    w1t = jnp.transpose(W1.reshape(SEQ, EMB, HID), (1, 0, 2)).reshape(EMB, -1)
    w1u = _k3a(embeddings, Wr, Wi, Wj, Wk, br[None], bi[None],
                  bj[None], bk[None], w1t)
    w1u4 = w1u.reshape(8, 3, 512, HID)
    cwsw = _k3b2(csa, w1u4)
    csign = jnp.asarray(CSIGN)[:, :, None, None]
    ssign = jnp.asarray(SSIGN)[:, :, None, None]
    t_tab = csign * cwsw[:, :5] + ssign * cwsw[:, 5:]
    tflat = t_tab.reshape(10 * FPAD, HID)
    out = _k4(vflat, tflat, b1[None], W2, b2[None])
    return jnp.transpose(out.reshape(B, D, PRED), (0, 2, 1))
